# Initial kernel scaffold; baseline (speedup 1.0000x reference)
import jax
import jax.numpy as jnp
from jax.experimental import pallas as pl


def kernel(token_ids, W):
    raise NotImplementedError("write your pallas kernel here")


# speedup vs baseline: 1.6854x; 1.6854x over previous
import functools

import jax
import jax.numpy as jnp
from jax import lax
from jax.experimental import pallas as pl
from jax.experimental.pallas import tpu as pltpu
from jax.experimental.pallas import tpu_sc as plsc

NUM_CORES = 2
NUM_SUBCORES = 16
NW = NUM_CORES * NUM_SUBCORES

CHUNK = 128


def _make_lookup(B, D):
    assert B % (NW * CHUNK) == 0
    per_w = B // NW
    n_chunks = per_w // CHUNK
    mesh = plsc.VectorSubcoreMesh(core_axis_name="c", subcore_axis_name="s")

    @functools.partial(
        pl.kernel,
        out_type=jax.ShapeDtypeStruct((B, D), jnp.float32),
        mesh=mesh,
        scratch_types=[
            pltpu.VMEM((n_chunks, CHUNK), jnp.int32),
            pltpu.VMEM((CHUNK, D), jnp.float32),
            pltpu.SemaphoreType.DMA,
        ],
        compiler_params=pltpu.CompilerParams(use_tc_tiling_on_sc=False),
    )
    def lookup(ids_hbm, w_hbm, out_hbm, idx_v, rows_v, sem):
        wid = lax.axis_index("s") * NUM_CORES + lax.axis_index("c")
        base = wid * per_w
        pltpu.sync_copy(ids_hbm.at[wid], idx_v)

        @pl.loop(0, n_chunks)
        def _chunk(j):
            pltpu.async_copy(w_hbm.at[idx_v.at[j]], rows_v, sem).wait()
            pltpu.sync_copy(rows_v, out_hbm.at[pl.ds(base + j * CHUNK, CHUNK)])

    return lookup


def kernel(token_ids, W):
    B_out = token_ids.shape
    D = W.shape[1]
    B = token_ids.size
    ids = token_ids.reshape(NW, B // (NW * CHUNK), CHUNK).astype(jnp.int32)
    out = _make_lookup(B, D)(ids, W)
    return out.reshape(*B_out, D)


# speedup vs baseline: 1.8759x; 1.1130x over previous
import functools

import jax
import jax.numpy as jnp
from jax import lax
from jax.experimental import pallas as pl
from jax.experimental.pallas import tpu as pltpu
from jax.experimental.pallas import tpu_sc as plsc

NUM_CORES = 2
NUM_SUBCORES = 16
NW = NUM_CORES * NUM_SUBCORES

CHUNK = 128
K = 6
S = 4
NBUF = K + S


def _make_lookup(B, D):
    assert B % (NW * CHUNK) == 0
    per_w = B // NW
    n = per_w // CHUNK
    n_steady = n - K - S
    assert n_steady > 0 and n_steady % NBUF == 0
    mesh = plsc.VectorSubcoreMesh(core_axis_name="c", subcore_axis_name="s")

    @functools.partial(
        pl.kernel,
        out_type=jax.ShapeDtypeStruct((B, D), jnp.float32),
        mesh=mesh,
        scratch_types=(
            [pltpu.VMEM((n, CHUNK), jnp.int32),
             pltpu.VMEM((NBUF, CHUNK, D), jnp.float32)]
            + [pltpu.SemaphoreType.DMA] * (2 * NBUF)
        ),
        compiler_params=pltpu.CompilerParams(use_tc_tiling_on_sc=False),
    )
    def lookup(ids_hbm, w_hbm, out_hbm, idx_v, rows_v, *sems):
        sem_g = sems[:NBUF]
        sem_s = sems[NBUF:]
        wid = lax.axis_index("s") * NUM_CORES + lax.axis_index("c")
        base = wid * per_w
        pltpu.sync_copy(ids_hbm.at[wid], idx_v)

        def start_gather(j, b):
            pltpu.async_copy(w_hbm.at[idx_v.at[j]], rows_v.at[b], sem_g[b])

        def wait_gather(j, b):
            pltpu.make_async_copy(
                w_hbm.at[idx_v.at[j]], rows_v.at[b], sem_g[b]).wait()

        def out_slice(j):
            return out_hbm.at[pl.ds(base + j * CHUNK, CHUNK)]

        def start_store(j, b):
            pltpu.async_copy(rows_v.at[b], out_slice(j), sem_s[b])

        def wait_store(j, b):
            pltpu.make_async_copy(rows_v.at[b], out_slice(j), sem_s[b]).wait()

        for j in range(K):
            start_gather(j, j % NBUF)

        for j in range(S):
            start_gather(j + K, (j + K) % NBUF)
            wait_gather(j, j % NBUF)
            start_store(j, j % NBUF)

        @pl.loop(0, n_steady // NBUF)
        def _group(g):
            for i in range(NBUF):
                b = (S + i) % NBUF
                j = S + g * NBUF + i
                bf = (S + i + K) % NBUF
                wait_store(j - S, bf)
                start_gather(j + K, bf)
                wait_gather(j, b)
                start_store(j, b)

        for i in range(K):
            j = n - K + i
            b = j % NBUF
            wait_store(j - S, (j + K) % NBUF)
            wait_gather(j, b)
            start_store(j, b)

        for i in range(S):
            j = n - S + i
            wait_store(j, j % NBUF)

    return lookup


def kernel(token_ids, W):
    B_out = token_ids.shape
    D = W.shape[1]
    B = token_ids.size
    ids = token_ids.reshape(NW, B // (NW * CHUNK), CHUNK).astype(jnp.int32)
    out = _make_lookup(B, D)(ids, W)
    return out.reshape(*B_out, D)
